# Initial kernel scaffold; baseline (speedup 1.0000x reference)
import jax
import jax.numpy as jnp
from jax.experimental import pallas as pl


def kernel(description, map_tensor, query, gt_coords):
    raise NotImplementedError("write your pallas kernel here")


# speedup vs baseline: 4.1921x; 4.1921x over previous
import functools

import jax
import jax.numpy as jnp
from jax.experimental import pallas as pl

_B, _H, _W, _E, _K = 4, 128, 128, 128, 32
_TOPK, _RAD, _NEIGH, _TAU = 8, 2, 1, 0.1
_HW = _H * _W


def _nms_kernel(x_ref, desc_ref, q_ref, vm_ref, coords_ref):
    x = x_ref[0]
    desc = desc_ref[0]
    q = q_ref[0]

    sumsq = jnp.sum(x * x, axis=1, keepdims=True)
    fm = x / jnp.maximum(jnp.sqrt(sumsq), 1e-12)

    dn = jnp.sqrt(jnp.sum(desc * desc, axis=1, keepdims=True))
    desc_h = desc / jnp.maximum(dn, 1e-12)
    qn = jnp.sqrt(jnp.sum(q * q, axis=1, keepdims=True))
    q_h = q / jnp.maximum(qn, 1e-12)
    qn2 = jnp.sqrt(jnp.sum(q_h * q_h, axis=1, keepdims=True))

    scores = jax.lax.dot_general(
        desc_h, fm, (((1,), (1,)), ((), ())),
        preferred_element_type=jnp.float32)

    val = jax.lax.dot_general(
        q_h, fm, (((1,), (1,)), ((), ())),
        precision=jax.lax.Precision.HIGHEST,
        preferred_element_type=jnp.float32)
    ones_row = jnp.ones((1, _E), dtype=jnp.float32)
    rnsq = jax.lax.dot_general(
        ones_row, fm * fm, (((1,), (1,)), ((), ())),
        precision=jax.lax.Precision.HIGHEST,
        preferred_element_type=jnp.float32)
    rn = jnp.sqrt(rnsq)

    pos = jax.lax.broadcasted_iota(jnp.int32, (1, _HW), 1)
    prow = pos // _W
    pcol = pos % _W

    neg_inf = jnp.float32(-jnp.inf)
    big = jnp.int32(2 ** 30)

    def step(_, carry):
        s, m = carry
        mx = jnp.max(s, axis=1, keepdims=True)
        cand = jnp.where(s >= mx, pos, big)
        idx = jnp.min(cand, axis=1, keepdims=True)
        row = idx // _W
        col = idx % _W
        dr = jnp.abs(prow - row)
        dc = jnp.abs(pcol - col)
        supp = (dr <= _RAD) & (dc <= _RAD)
        s = jnp.where(supp, neg_inf, s)
        nb = ((dr <= _NEIGH) & (dc <= _NEIGH)).astype(jnp.float32)
        m = jnp.maximum(m, jnp.max(nb, axis=0, keepdims=True))
        return s, m

    mask0 = jnp.zeros((1, _HW), dtype=jnp.float32)
    _, mask = jax.lax.fori_loop(0, _TOPK, step, (scores, mask0))

    num = mask * val
    den = jnp.maximum(mask * rn * qn2, 1e-8)
    vm = num / den
    vm_ref[0] = vm

    mv = jnp.max(vm, axis=1, keepdims=True)
    p = jnp.exp((vm - mv) / _TAU)
    z = jnp.sum(p, axis=1, keepdims=True)
    ey = jnp.sum(p * prow.astype(jnp.float32), axis=1, keepdims=True) / z
    ex = jnp.sum(p * pcol.astype(jnp.float32), axis=1, keepdims=True) / z
    coords_ref[0] = jnp.concatenate([ey, ex], axis=1)


@jax.jit
def kernel(description, map_tensor, query, gt_coords):
    del gt_coords
    x = map_tensor.reshape(_B, _HW, _E)
    qr = query.reshape(_B, 1, _E)

    vm, coords = pl.pallas_call(
        _nms_kernel,
        grid=(_B,),
        in_specs=[
            pl.BlockSpec((1, _HW, _E), lambda b: (b, 0, 0)),
            pl.BlockSpec((1, _K, _E), lambda b: (b, 0, 0)),
            pl.BlockSpec((1, 1, _E), lambda b: (b, 0, 0)),
        ],
        out_specs=[
            pl.BlockSpec((1, 1, _HW), lambda b: (b, 0, 0)),
            pl.BlockSpec((1, 1, 2), lambda b: (b, 0, 0)),
        ],
        out_shape=[
            jax.ShapeDtypeStruct((_B, 1, _HW), jnp.float32),
            jax.ShapeDtypeStruct((_B, 1, 2), jnp.float32),
        ],
    )(x, description, qr)

    return vm.reshape(_B, _H, _W, 1), coords.reshape(_B, 2)


# speedup vs baseline: 5.0001x; 1.1927x over previous
import jax
import jax.numpy as jnp
from jax.experimental import pallas as pl
from jax.experimental.pallas import tpu as pltpu

_B, _H, _W, _E, _K = 4, 128, 128, 128, 32
_TOPK, _RAD, _NEIGH, _TAU = 8, 2, 1, 0.1
_HW = _H * _W
_BK = _B * _K


def _fused_kernel(x_ref, desc_ref, q_ref, vm_ref, coords_ref,
                  s_ref, val_ref, rnq_ref):
    b = pl.program_id(0)

    @pl.when(b < _B)
    def _scores_phase():
        x = x_ref[0]
        desc = desc_ref[0]
        q = q_ref[0]

        sumsq = jnp.sum(x * x, axis=1, keepdims=True)
        fm = x / jnp.maximum(jnp.sqrt(sumsq), 1e-12)

        dn = jnp.sqrt(jnp.sum(desc * desc, axis=1, keepdims=True))
        desc_h = desc / jnp.maximum(dn, 1e-12)
        qn = jnp.sqrt(jnp.sum(q * q, axis=1, keepdims=True))
        q_h = q / jnp.maximum(qn, 1e-12)
        qn2 = jnp.sqrt(jnp.sum(q_h * q_h, axis=1, keepdims=True))

        s_ref[pl.ds(b * _K, _K)] = jax.lax.dot_general(
            desc_h, fm, (((1,), (1,)), ((), ())),
            preferred_element_type=jnp.float32)

        val_ref[b] = jax.lax.dot_general(
            q_h, fm, (((1,), (1,)), ((), ())),
            precision=jax.lax.Precision.HIGHEST,
            preferred_element_type=jnp.float32)
        ones_row = jnp.ones((1, _E), dtype=jnp.float32)
        rnsq = jax.lax.dot_general(
            ones_row, fm * fm, (((1,), (1,)), ((), ())),
            precision=jax.lax.Precision.HIGHEST,
            preferred_element_type=jnp.float32)
        rnq_ref[b] = jnp.sqrt(rnsq) * qn2

    @pl.when(b == _B)
    def _nms_phase():
        posf = jax.lax.broadcasted_iota(
            jnp.int32, (1, _HW), 1).astype(jnp.float32)
        prow = jnp.floor(posf * (1.0 / _W))
        pcol = posf - prow * _W

        neg_inf = jnp.float32(-jnp.inf)
        bigf = jnp.float32(2.0 ** 30)

        def step(_, carry):
            mx, mask = carry
            s = s_ref[...]
            cand = jnp.where(s >= mx, posf, bigf)
            idx = jnp.min(cand, axis=1, keepdims=True)
            row = jnp.floor(idx * (1.0 / _W))
            col = idx - row * _W
            dr = jnp.abs(prow - row)
            dc = jnp.abs(pcol - col)
            supp = (dr <= _RAD) & (dc <= _RAD)
            s_new = jnp.where(supp, neg_inf, s)
            s_ref[...] = s_new
            mx = jnp.max(s_new, axis=1, keepdims=True)
            nb = ((dr <= _NEIGH) & (dc <= _NEIGH)).astype(jnp.float32)
            m4 = jnp.concatenate(
                [jnp.max(nb[i * _K:(i + 1) * _K], axis=0, keepdims=True)
                 for i in range(_B)], axis=0)
            return mx, jnp.maximum(mask, m4)

        mx0 = jnp.max(s_ref[...], axis=1, keepdims=True)
        mask0 = jnp.zeros((_B, _HW), dtype=jnp.float32)
        _, mask = jax.lax.fori_loop(0, _TOPK, step, (mx0, mask0))

        for i in range(_B):
            bm = mask[i:i + 1]
            num = bm * val_ref[i]
            den = jnp.maximum(bm * rnq_ref[i], 1e-8)
            vm = num / den
            vm_ref[i] = vm
            mv = jnp.max(vm, axis=1, keepdims=True)
            p = jnp.exp((vm - mv) / _TAU)
            z = jnp.sum(p, axis=1, keepdims=True)
            ey = jnp.sum(p * prow, axis=1, keepdims=True) / z
            ex = jnp.sum(p * pcol, axis=1, keepdims=True) / z
            coords_ref[i] = jnp.concatenate([ey, ex], axis=1)


@jax.jit
def kernel(description, map_tensor, query, gt_coords):
    del gt_coords
    x = map_tensor.reshape(_B, _HW, _E)
    qr = query.reshape(_B, 1, _E)

    vm, coords = pl.pallas_call(
        _fused_kernel,
        grid=(_B + 1,),
        in_specs=[
            pl.BlockSpec((1, _HW, _E), lambda b: (jnp.minimum(b, _B - 1), 0, 0)),
            pl.BlockSpec((1, _K, _E), lambda b: (jnp.minimum(b, _B - 1), 0, 0)),
            pl.BlockSpec((1, 1, _E), lambda b: (jnp.minimum(b, _B - 1), 0, 0)),
        ],
        out_specs=[
            pl.BlockSpec((_B, 1, _HW), lambda b: (0, 0, 0)),
            pl.BlockSpec((_B, 1, 2), lambda b: (0, 0, 0)),
        ],
        out_shape=[
            jax.ShapeDtypeStruct((_B, 1, _HW), jnp.float32),
            jax.ShapeDtypeStruct((_B, 1, 2), jnp.float32),
        ],
        scratch_shapes=[
            pltpu.VMEM((_BK, _HW), jnp.float32),
            pltpu.VMEM((_B, 1, _HW), jnp.float32),
            pltpu.VMEM((_B, 1, _HW), jnp.float32),
        ],
    )(x, description, qr)

    return vm.reshape(_B, _H, _W, 1), coords.reshape(_B, 2)
